# baseline (speedup 1.0000x reference)
import functools

import jax
import jax.numpy as jnp
from jax import lax
from jax.experimental import pallas as pl
from jax.experimental.pallas import tpu as pltpu
from jax.experimental.pallas import tpu_sc as plsc

N = 10000
E = 320000
NC = 2
NS = 16
NW = NC * NS
CH = 125
CHUNKS_PROP = 160
GRP = 40
ROW_STEP = 624
ROW_SPAN = 640
W1H = 80
W2P = 272
W2H = W2P // 2
BN_S = 0.9995004

_mesh = plsc.VectorSubcoreMesh(core_axis_name="c", subcore_axis_name="s")


NPAD = 10240
DEG_CH = 50
DEG_CHUNKS = 200


@functools.partial(
    pl.kernel,
    out_type=jax.ShapeDtypeStruct((NC, NPAD, 16), jnp.float32),
    mesh=_mesh,
    compiler_params=pltpu.CompilerParams(use_tc_tiling_on_sc=False),
    scratch_types=[
        pltpu.VMEM((DEG_CHUNKS, DEG_CH), jnp.int32),
        pltpu.VMEM((64, 16), jnp.float32),
        pltpu.SemaphoreType.DMA,
        pltpu.VMEM_SHARED((NPAD, 16), jnp.float32),
    ],
)
def _deg_kernel(e50_hbm, zeros_hbm, ones_hbm, out_hbm, dst_v, ones_v, sem, dacc):
    c = lax.axis_index("c")
    s = lax.axis_index("s")
    wid = c * NS + s
    wb = pl.multiple_of(wid * DEG_CHUNKS, 8)
    pltpu.sync_copy(e50_hbm.at[1].at[pl.ds(wb, DEG_CHUNKS)], dst_v)
    pltpu.sync_copy(ones_hbm, ones_v)
    dbase = pl.multiple_of(s * (NPAD // NS), 8)
    pltpu.sync_copy(zeros_hbm.at[pl.ds(dbase, NPAD // NS)],
                    dacc.at[pl.ds(dbase, NPAD // NS)])
    plsc.subcore_barrier()

    def body(j, carry):
        pltpu.async_copy(ones_v.at[pl.ds(0, DEG_CH)], dacc.at[dst_v.at[j]],
                         sem, add=True)
        return carry

    lax.fori_loop(0, DEG_CHUNKS, body, 0)

    def drain(j, carry):
        pltpu.make_async_copy(ones_v.at[pl.ds(0, DEG_CH)],
                              dacc.at[dst_v.at[j]], sem).wait()
        return carry

    lax.fori_loop(0, DEG_CHUNKS, drain, 0)
    plsc.subcore_barrier()
    pltpu.sync_copy(dacc.at[pl.ds(dbase, NPAD // NS)],
                    out_hbm.at[c].at[pl.ds(dbase, NPAD // NS)])


def _make_propd(width, ch, grp, depth, lead):
    chunks = E // (NS * ch)
    nblk = grp // depth

    @functools.partial(
        pl.kernel,
        out_type=(jax.ShapeDtypeStruct((N, width), jnp.float32),
                  jax.ShapeDtypeStruct((N, width), jnp.float32)),
        mesh=_mesh,
        compiler_params=pltpu.CompilerParams(use_tc_tiling_on_sc=False),
        scratch_types=[
            pltpu.VMEM((grp, ch), jnp.int32),
            pltpu.VMEM((grp, ch), jnp.int32),
            [pltpu.VMEM((ch, width), jnp.float32)] * depth,
            [pltpu.SemaphoreType.DMA] * depth,
            [pltpu.SemaphoreType.DMA] * depth,
            pltpu.VMEM_SHARED((N, width), jnp.float32),
        ],
    )
    def prop_k(e4_hbm, v0_hbm, v1_hbm, out0_hbm, out1_hbm,
               src_v, dst_v, bufs, gsems, ssems, acc):
        c = lax.axis_index("c")
        s = lax.axis_index("s")
        base = pl.multiple_of(s * ROW_STEP, 8)

        def run(v_hbm, out_hbm):
            pltpu.sync_copy(v_hbm.at[pl.ds(base, ROW_SPAN)],
                            acc.at[pl.ds(base, ROW_SPAN)])
            plsc.subcore_barrier()

            def group(g, carry):
                tb = pl.multiple_of(s * chunks + g * grp, 8)
                pltpu.sync_copy(e4_hbm.at[0].at[pl.ds(tb, grp)], src_v)
                pltpu.sync_copy(e4_hbm.at[1].at[pl.ds(tb, grp)], dst_v)
                for t in range(lead):
                    pltpu.async_copy(v_hbm.at[src_v.at[t]], bufs[t], gsems[t])

                def block(qb, carry2):
                    j0 = depth * qb
                    for k in range(depth):
                        j = j0 + k
                        kg = (k + lead) % depth
                        jm = j - (depth - lead)
                        if k < depth - lead:
                            @pl.when(qb > 0)
                            def _():
                                pltpu.make_async_copy(
                                    bufs[kg],
                                    acc.at[dst_v.at[jnp.maximum(jm, 0)]],
                                    ssems[kg]).wait()
                            pltpu.async_copy(v_hbm.at[src_v.at[j + lead]],
                                             bufs[kg], gsems[kg])
                        else:
                            pltpu.make_async_copy(
                                bufs[kg], acc.at[dst_v.at[jm]],
                                ssems[kg]).wait()

                            @pl.when(qb < nblk - 1)
                            def _():
                                pltpu.async_copy(v_hbm.at[src_v.at[j + lead]],
                                                 bufs[kg], gsems[kg])
                        pltpu.make_async_copy(v_hbm.at[src_v.at[j]],
                                              bufs[k], gsems[k]).wait()
                        pltpu.async_copy(bufs[k], acc.at[dst_v.at[j]],
                                         ssems[k], add=True)
                    return carry2

                lax.fori_loop(0, nblk, block, 0)
                for t in range(depth - lead):
                    ke = (lead + t) % depth
                    pltpu.make_async_copy(
                        bufs[ke], acc.at[dst_v.at[grp - (depth - lead) + t]],
                        ssems[ke]).wait()
                return carry

            lax.fori_loop(0, chunks // grp, group, 0)
            plsc.subcore_barrier()
            pltpu.sync_copy(acc.at[pl.ds(base, ROW_SPAN)],
                            out_hbm.at[pl.ds(base, ROW_SPAN)])

        @pl.when(c == 0)
        def _():
            run(v0_hbm, out0_hbm)

        @pl.when(c == 1)
        def _():
            run(v1_hbm, out1_hbm)

    return prop_k


def _make_prop(width, ch, grp):
    chunks = E // (NS * ch)

    @functools.partial(
        pl.kernel,
        out_type=(jax.ShapeDtypeStruct((N, width), jnp.float32),
                  jax.ShapeDtypeStruct((N, width), jnp.float32)),
        mesh=_mesh,
        compiler_params=pltpu.CompilerParams(use_tc_tiling_on_sc=False),
        scratch_types=[
            pltpu.VMEM((grp, ch), jnp.int32),
            pltpu.VMEM((grp, ch), jnp.int32),
            pltpu.VMEM((ch, width), jnp.float32),
            pltpu.VMEM((ch, width), jnp.float32),
            pltpu.SemaphoreType.DMA,
            pltpu.SemaphoreType.DMA,
            pltpu.SemaphoreType.DMA,
            pltpu.SemaphoreType.DMA,
            pltpu.VMEM_SHARED((N, width), jnp.float32),
        ],
    )
    def prop_k(e4_hbm, v0_hbm, v1_hbm, out0_hbm, out1_hbm,
               src_v, dst_v, bufa, bufb, sga, sgb, ssa, ssb, acc):
        c = lax.axis_index("c")
        s = lax.axis_index("s")

        base = pl.multiple_of(s * ROW_STEP, 8)

        def run(v_hbm, out_hbm):
            pltpu.sync_copy(v_hbm.at[pl.ds(base, ROW_SPAN)],
                            acc.at[pl.ds(base, ROW_SPAN)])
            plsc.subcore_barrier()

            def group(g, carry):
                gb = g * grp
                tb = pl.multiple_of(s * chunks + gb, 8)
                pltpu.sync_copy(e4_hbm.at[0].at[pl.ds(tb, grp)], src_v)
                pltpu.sync_copy(e4_hbm.at[1].at[pl.ds(tb, grp)], dst_v)
                pltpu.async_copy(v_hbm.at[src_v.at[0]], bufa, sga)

                def pair(p, carry2):
                    j0 = 2 * p
                    j1 = j0 + 1
                    gb_desc = pltpu.async_copy(v_hbm.at[src_v.at[j1]], bufb, sgb)
                    pltpu.make_async_copy(v_hbm.at[src_v.at[j0]], bufa, sga).wait()
                    sa_desc = pltpu.async_copy(bufa, acc.at[dst_v.at[j0]], ssa,
                                               add=True)
                    gb_desc.wait()
                    sb_desc = pltpu.async_copy(bufb, acc.at[dst_v.at[j1]], ssb,
                                               add=True)
                    sa_desc.wait()

                    @pl.when(p < grp // 2 - 1)
                    def _():
                        pltpu.async_copy(v_hbm.at[src_v.at[j0 + 2]], bufa, sga)

                    sb_desc.wait()
                    return carry2

                lax.fori_loop(0, grp // 2, pair, 0)
                return carry

            lax.fori_loop(0, chunks // grp, group, 0)
            plsc.subcore_barrier()
            pltpu.sync_copy(acc.at[pl.ds(base, ROW_SPAN)],
                            out_hbm.at[pl.ds(base, ROW_SPAN)])

        @pl.when(c == 0)
        def _():
            run(v0_hbm, out0_hbm)

        @pl.when(c == 1)
        def _():
            run(v1_hbm, out1_hbm)

    return prop_k


_prop80 = _make_propd(W1H, 50, 40, 5, 2)
_prop144 = _make_propd(W2H, 50, 40, 5, 2)


_BS = 2000


def _scale_body(x_ref, degp_ref, v0_ref, v1_ref):
    deg = degp_ref[0, :, 0] + degp_ref[1, :, 0] + 1.0
    dinv = lax.rsqrt(deg)
    v = x_ref[...] * dinv[:, None]
    v0_ref[...] = v[:, :W1H]
    v1_ref[...] = v[:, W1H:]


def _tc_scale(x, degp):
    return pl.pallas_call(
        _scale_body,
        grid=(N // _BS,),
        in_specs=[
            pl.BlockSpec((_BS, 160), lambda i: (i, 0)),
            pl.BlockSpec((NC, _BS, 16), lambda i: (0, i, 0)),
        ],
        out_specs=[
            pl.BlockSpec((_BS, W1H), lambda i: (i, 0)),
            pl.BlockSpec((_BS, W1H), lambda i: (i, 0)),
        ],
        out_shape=[jax.ShapeDtypeStruct((N, W1H), jnp.float32)] * 2,
    )(x, degp)


def _mid_body(s0_ref, s1_ref, degp_ref, W1_ref, b1_ref, g1_ref, be1_ref,
              W2_ref, h0_ref, h1_ref):
    deg = degp_ref[0, :, 0] + degp_ref[1, :, 0] + 1.0
    dinv = lax.rsqrt(deg)
    p = jnp.concatenate([s0_ref[...], s1_ref[...]], axis=1) * dinv[:, None]
    z1 = jnp.dot(p, W1_ref[...], preferred_element_type=jnp.float32)
    z1 = z1 + b1_ref[...][None, :]
    x1 = jax.nn.relu(z1) * (g1_ref[...] * BN_S)[None, :] + be1_ref[...][None, :]
    x1 = jax.nn.sigmoid(x1)
    h2 = jnp.dot(x1, W2_ref[...], preferred_element_type=jnp.float32)
    h2 = h2 * dinv[:, None]
    h0_ref[...] = h2[:, :W2H]
    h1_ref[...] = jnp.concatenate(
        [h2[:, W2H:], jnp.zeros((_BS, W2P - 268), jnp.float32)], axis=1)


def _tc_mid(s0, s1, degp, W1, b1, g1, be1, W2):
    return pl.pallas_call(
        _mid_body,
        grid=(N // _BS,),
        in_specs=[
            pl.BlockSpec((_BS, W1H), lambda i: (i, 0)),
            pl.BlockSpec((_BS, W1H), lambda i: (i, 0)),
            pl.BlockSpec((NC, _BS, 16), lambda i: (0, i, 0)),
            pl.BlockSpec((160, 536), lambda i: (0, 0)),
            pl.BlockSpec((536,), lambda i: (0,)),
            pl.BlockSpec((536,), lambda i: (0,)),
            pl.BlockSpec((536,), lambda i: (0,)),
            pl.BlockSpec((536, 268), lambda i: (0, 0)),
        ],
        out_specs=[
            pl.BlockSpec((_BS, W2H), lambda i: (i, 0)),
            pl.BlockSpec((_BS, W2H), lambda i: (i, 0)),
        ],
        out_shape=[jax.ShapeDtypeStruct((N, W2H), jnp.float32)] * 2,
    )(s0, s1, degp, W1, b1, g1, be1, W2)


def _final_body(t0_ref, t1_ref, degp_ref, b2_ref, g2_ref, be2_ref, out_ref):
    i = pl.program_id(0)
    deg = degp_ref[0, :, 0] + degp_ref[1, :, 0] + 1.0
    dinv = lax.rsqrt(deg)
    p = jnp.concatenate([t0_ref[...], t1_ref[...]], axis=1) * dinv[:, None]
    zpad = jnp.zeros((W2P - 268,), jnp.float32)
    b2 = jnp.concatenate([b2_ref[...], zpad])
    g2 = jnp.concatenate([g2_ref[...], zpad])
    be2 = jnp.concatenate([be2_ref[...], zpad])
    z2 = p + b2[None, :]
    x2 = jax.nn.relu(z2) * (g2 * BN_S)[None, :] + be2[None, :]
    x2 = jax.nn.sigmoid(x2)
    col = lax.broadcasted_iota(jnp.int32, x2.shape, 1)
    x2 = jnp.where(col < 268, x2, 0.0)
    g = lax.dot_general(x2, x2, (((0,), (0,)), ((), ())),
                        preferred_element_type=jnp.float32)[:268, :268]

    @pl.when(i == 0)
    def _():
        out_ref[...] = g

    @pl.when(i > 0)
    def _():
        out_ref[...] += g


def _tc_final(t0, t1, degp, b2, g2, be2):
    return pl.pallas_call(
        _final_body,
        grid=(N // _BS,),
        in_specs=[
            pl.BlockSpec((_BS, W2H), lambda i: (i, 0)),
            pl.BlockSpec((_BS, W2H), lambda i: (i, 0)),
            pl.BlockSpec((NC, _BS, 16), lambda i: (0, i, 0)),
            pl.BlockSpec((268,), lambda i: (0,)),
            pl.BlockSpec((268,), lambda i: (0,)),
            pl.BlockSpec((268,), lambda i: (0,)),
        ],
        out_specs=pl.BlockSpec((268, 268), lambda i: (0, 0)),
        out_shape=jax.ShapeDtypeStruct((268, 268), jnp.float32),
    )(t0, t1, degp, b2, g2, be2)


def kernel(x, edge_index, edge_attr, W1, b1, gamma1, beta1, W2, b2, gamma2,
           beta2):
    xs = x.reshape(N, 160)
    e50 = edge_index.reshape(2, E // 50, 50)
    zeros_n = jnp.zeros((NPAD, 16), jnp.float32)
    ones_n = jnp.ones((64, 16), jnp.float32)

    degp = _deg_kernel(e50, zeros_n, ones_n)
    v0, v1 = _tc_scale(xs, degp)
    s0, s1 = _prop80(e50, v0, v1)
    h0, h1 = _tc_mid(s0, s1, degp, W1, b1, gamma1, beta1, W2)
    t0, t1 = _prop144(e50, h0, h1)
    return _tc_final(t0, t1, degp, b2, gamma2, beta2)


# speedup vs baseline: 1.0869x; 1.0869x over previous
import functools

import jax
import jax.numpy as jnp
from jax import lax
from jax.experimental import pallas as pl
from jax.experimental.pallas import tpu as pltpu
from jax.experimental.pallas import tpu_sc as plsc

N = 10000
E = 320000
NC = 2
NS = 16
NW = NC * NS
CH = 125
CHUNKS_PROP = 160
GRP = 40
ROW_STEP = 624
ROW_SPAN = 640
W1H = 80
W2P = 272
W2H = W2P // 2
BN_S = 0.9995004

_mesh = plsc.VectorSubcoreMesh(core_axis_name="c", subcore_axis_name="s")


NPAD = 10240
DEG_CH = 125
DEG_CHUNKS = 80


@functools.partial(
    pl.kernel,
    out_type=jax.ShapeDtypeStruct((NC, NPAD, 16), jnp.float32),
    mesh=_mesh,
    compiler_params=pltpu.CompilerParams(use_tc_tiling_on_sc=False),
    scratch_types=[
        pltpu.VMEM((DEG_CHUNKS, DEG_CH), jnp.int32),
        pltpu.VMEM((128, 16), jnp.float32),
        pltpu.SemaphoreType.DMA,
        pltpu.VMEM_SHARED((NPAD, 16), jnp.float32),
    ],
)
def _deg_kernel(e4_hbm, zeros_hbm, ones_hbm, out_hbm, dst_v, ones_v, sem, dacc):
    c = lax.axis_index("c")
    s = lax.axis_index("s")
    wid = c * NS + s
    wb = pl.multiple_of(wid * DEG_CHUNKS, 8)
    pltpu.sync_copy(e4_hbm.at[1].at[pl.ds(wb, DEG_CHUNKS)], dst_v)
    pltpu.sync_copy(ones_hbm, ones_v)
    dbase = pl.multiple_of(s * (NPAD // NS), 8)
    pltpu.sync_copy(zeros_hbm.at[pl.ds(dbase, NPAD // NS)],
                    dacc.at[pl.ds(dbase, NPAD // NS)])
    plsc.subcore_barrier()

    def body(j, carry):
        pltpu.async_copy(ones_v.at[pl.ds(0, DEG_CH)], dacc.at[dst_v.at[j]],
                         sem, add=True)
        return carry

    lax.fori_loop(0, DEG_CHUNKS, body, 0)

    def drain(j, carry):
        pltpu.make_async_copy(ones_v.at[pl.ds(0, DEG_CH)],
                              dacc.at[dst_v.at[j]], sem).wait()
        return carry

    lax.fori_loop(0, DEG_CHUNKS, drain, 0)
    plsc.subcore_barrier()
    pltpu.sync_copy(dacc.at[pl.ds(dbase, NPAD // NS)],
                    out_hbm.at[c].at[pl.ds(dbase, NPAD // NS)])


def _make_propd(width, ch, grp, depth, lead):
    chunks = E // (NS * ch)
    nblk = grp // depth

    @functools.partial(
        pl.kernel,
        out_type=(jax.ShapeDtypeStruct((N, width), jnp.float32),
                  jax.ShapeDtypeStruct((N, width), jnp.float32)),
        mesh=_mesh,
        compiler_params=pltpu.CompilerParams(use_tc_tiling_on_sc=False),
        scratch_types=[
            pltpu.VMEM((grp, ch), jnp.int32),
            pltpu.VMEM((grp, ch), jnp.int32),
            [pltpu.VMEM((ch, width), jnp.float32)] * depth,
            [pltpu.SemaphoreType.DMA] * depth,
            [pltpu.SemaphoreType.DMA] * depth,
            pltpu.VMEM_SHARED((N, width), jnp.float32),
        ],
    )
    def prop_k(e4_hbm, v0_hbm, v1_hbm, out0_hbm, out1_hbm,
               src_v, dst_v, bufs, gsems, ssems, acc):
        c = lax.axis_index("c")
        s = lax.axis_index("s")
        base = pl.multiple_of(s * ROW_STEP, 8)

        def run(v_hbm, out_hbm):
            pltpu.sync_copy(v_hbm.at[pl.ds(base, ROW_SPAN)],
                            acc.at[pl.ds(base, ROW_SPAN)])
            plsc.subcore_barrier()

            def group(g, carry):
                tb = pl.multiple_of(s * chunks + g * grp, 8)
                pltpu.sync_copy(e4_hbm.at[0].at[pl.ds(tb, grp)], src_v)
                pltpu.sync_copy(e4_hbm.at[1].at[pl.ds(tb, grp)], dst_v)
                for t in range(lead):
                    pltpu.async_copy(v_hbm.at[src_v.at[t]], bufs[t], gsems[t])

                def block(qb, carry2):
                    j0 = depth * qb
                    for k in range(depth):
                        j = j0 + k
                        kg = (k + lead) % depth
                        jm = j - (depth - lead)
                        if k < depth - lead:
                            @pl.when(qb > 0)
                            def _():
                                pltpu.make_async_copy(
                                    bufs[kg],
                                    acc.at[dst_v.at[jnp.maximum(jm, 0)]],
                                    ssems[kg]).wait()
                            pltpu.async_copy(v_hbm.at[src_v.at[j + lead]],
                                             bufs[kg], gsems[kg])
                        else:
                            pltpu.make_async_copy(
                                bufs[kg], acc.at[dst_v.at[jm]],
                                ssems[kg]).wait()

                            @pl.when(qb < nblk - 1)
                            def _():
                                pltpu.async_copy(v_hbm.at[src_v.at[j + lead]],
                                                 bufs[kg], gsems[kg])
                        pltpu.make_async_copy(v_hbm.at[src_v.at[j]],
                                              bufs[k], gsems[k]).wait()
                        pltpu.async_copy(bufs[k], acc.at[dst_v.at[j]],
                                         ssems[k], add=True)
                    return carry2

                lax.fori_loop(0, nblk, block, 0)
                for t in range(depth - lead):
                    ke = (lead + t) % depth
                    pltpu.make_async_copy(
                        bufs[ke], acc.at[dst_v.at[grp - (depth - lead) + t]],
                        ssems[ke]).wait()
                return carry

            lax.fori_loop(0, chunks // grp, group, 0)
            plsc.subcore_barrier()
            pltpu.sync_copy(acc.at[pl.ds(base, ROW_SPAN)],
                            out_hbm.at[pl.ds(base, ROW_SPAN)])

        @pl.when(c == 0)
        def _():
            run(v0_hbm, out0_hbm)

        @pl.when(c == 1)
        def _():
            run(v1_hbm, out1_hbm)

    return prop_k


def _make_prop(width, ch, grp):
    chunks = E // (NS * ch)

    @functools.partial(
        pl.kernel,
        out_type=(jax.ShapeDtypeStruct((N, width), jnp.float32),
                  jax.ShapeDtypeStruct((N, width), jnp.float32)),
        mesh=_mesh,
        compiler_params=pltpu.CompilerParams(use_tc_tiling_on_sc=False),
        scratch_types=[
            pltpu.VMEM((grp, ch), jnp.int32),
            pltpu.VMEM((grp, ch), jnp.int32),
            pltpu.VMEM((ch, width), jnp.float32),
            pltpu.VMEM((ch, width), jnp.float32),
            pltpu.SemaphoreType.DMA,
            pltpu.SemaphoreType.DMA,
            pltpu.SemaphoreType.DMA,
            pltpu.SemaphoreType.DMA,
            pltpu.VMEM_SHARED((N, width), jnp.float32),
        ],
    )
    def prop_k(e4_hbm, v0_hbm, v1_hbm, out0_hbm, out1_hbm,
               src_v, dst_v, bufa, bufb, sga, sgb, ssa, ssb, acc):
        c = lax.axis_index("c")
        s = lax.axis_index("s")

        base = pl.multiple_of(s * ROW_STEP, 8)

        def run(v_hbm, out_hbm):
            pltpu.sync_copy(v_hbm.at[pl.ds(base, ROW_SPAN)],
                            acc.at[pl.ds(base, ROW_SPAN)])
            plsc.subcore_barrier()

            def group(g, carry):
                gb = g * grp
                tb = pl.multiple_of(s * chunks + gb, 8)
                pltpu.sync_copy(e4_hbm.at[0].at[pl.ds(tb, grp)], src_v)
                pltpu.sync_copy(e4_hbm.at[1].at[pl.ds(tb, grp)], dst_v)
                pltpu.async_copy(v_hbm.at[src_v.at[0]], bufa, sga)

                def pair(p, carry2):
                    j0 = 2 * p
                    j1 = j0 + 1
                    gb_desc = pltpu.async_copy(v_hbm.at[src_v.at[j1]], bufb, sgb)
                    pltpu.make_async_copy(v_hbm.at[src_v.at[j0]], bufa, sga).wait()
                    sa_desc = pltpu.async_copy(bufa, acc.at[dst_v.at[j0]], ssa,
                                               add=True)
                    gb_desc.wait()
                    sb_desc = pltpu.async_copy(bufb, acc.at[dst_v.at[j1]], ssb,
                                               add=True)
                    sa_desc.wait()

                    @pl.when(p < grp // 2 - 1)
                    def _():
                        pltpu.async_copy(v_hbm.at[src_v.at[j0 + 2]], bufa, sga)

                    sb_desc.wait()
                    return carry2

                lax.fori_loop(0, grp // 2, pair, 0)
                return carry

            lax.fori_loop(0, chunks // grp, group, 0)
            plsc.subcore_barrier()
            pltpu.sync_copy(acc.at[pl.ds(base, ROW_SPAN)],
                            out_hbm.at[pl.ds(base, ROW_SPAN)])

        @pl.when(c == 0)
        def _():
            run(v0_hbm, out0_hbm)

        @pl.when(c == 1)
        def _():
            run(v1_hbm, out1_hbm)

    return prop_k


_prop80 = _make_propd(W1H, 125, 40, 5, 2)
_prop144 = _make_propd(W2H, 50, 40, 5, 2)


_BS = 2000


def _scale_body(x_ref, degp_ref, v0_ref, v1_ref):
    deg = degp_ref[0, :, 0] + degp_ref[1, :, 0] + 1.0
    dinv = lax.rsqrt(deg)
    v = x_ref[...] * dinv[:, None]
    v0_ref[...] = v[:, :W1H]
    v1_ref[...] = v[:, W1H:]


def _tc_scale(x, degp):
    return pl.pallas_call(
        _scale_body,
        grid=(N // _BS,),
        in_specs=[
            pl.BlockSpec((_BS, 160), lambda i: (i, 0)),
            pl.BlockSpec((NC, _BS, 16), lambda i: (0, i, 0)),
        ],
        out_specs=[
            pl.BlockSpec((_BS, W1H), lambda i: (i, 0)),
            pl.BlockSpec((_BS, W1H), lambda i: (i, 0)),
        ],
        out_shape=[jax.ShapeDtypeStruct((N, W1H), jnp.float32)] * 2,
    )(x, degp)


def _mid_body(s0_ref, s1_ref, degp_ref, W1_ref, b1_ref, g1_ref, be1_ref,
              W2_ref, h0_ref, h1_ref):
    deg = degp_ref[0, :, 0] + degp_ref[1, :, 0] + 1.0
    dinv = lax.rsqrt(deg)
    p = jnp.concatenate([s0_ref[...], s1_ref[...]], axis=1) * dinv[:, None]
    z1 = jnp.dot(p, W1_ref[...], preferred_element_type=jnp.float32)
    z1 = z1 + b1_ref[...][None, :]
    x1 = jax.nn.relu(z1) * (g1_ref[...] * BN_S)[None, :] + be1_ref[...][None, :]
    x1 = jax.nn.sigmoid(x1)
    h2 = jnp.dot(x1, W2_ref[...], preferred_element_type=jnp.float32)
    h2 = h2 * dinv[:, None]
    h0_ref[...] = h2[:, :W2H]
    h1_ref[...] = jnp.concatenate(
        [h2[:, W2H:], jnp.zeros((_BS, W2P - 268), jnp.float32)], axis=1)


def _tc_mid(s0, s1, degp, W1, b1, g1, be1, W2):
    return pl.pallas_call(
        _mid_body,
        grid=(N // _BS,),
        in_specs=[
            pl.BlockSpec((_BS, W1H), lambda i: (i, 0)),
            pl.BlockSpec((_BS, W1H), lambda i: (i, 0)),
            pl.BlockSpec((NC, _BS, 16), lambda i: (0, i, 0)),
            pl.BlockSpec((160, 536), lambda i: (0, 0)),
            pl.BlockSpec((536,), lambda i: (0,)),
            pl.BlockSpec((536,), lambda i: (0,)),
            pl.BlockSpec((536,), lambda i: (0,)),
            pl.BlockSpec((536, 268), lambda i: (0, 0)),
        ],
        out_specs=[
            pl.BlockSpec((_BS, W2H), lambda i: (i, 0)),
            pl.BlockSpec((_BS, W2H), lambda i: (i, 0)),
        ],
        out_shape=[jax.ShapeDtypeStruct((N, W2H), jnp.float32)] * 2,
    )(s0, s1, degp, W1, b1, g1, be1, W2)


def _final_body(t0_ref, t1_ref, degp_ref, b2_ref, g2_ref, be2_ref, out_ref):
    i = pl.program_id(0)
    deg = degp_ref[0, :, 0] + degp_ref[1, :, 0] + 1.0
    dinv = lax.rsqrt(deg)
    p = jnp.concatenate([t0_ref[...], t1_ref[...]], axis=1) * dinv[:, None]
    zpad = jnp.zeros((W2P - 268,), jnp.float32)
    b2 = jnp.concatenate([b2_ref[...], zpad])
    g2 = jnp.concatenate([g2_ref[...], zpad])
    be2 = jnp.concatenate([be2_ref[...], zpad])
    z2 = p + b2[None, :]
    x2 = jax.nn.relu(z2) * (g2 * BN_S)[None, :] + be2[None, :]
    x2 = jax.nn.sigmoid(x2)
    col = lax.broadcasted_iota(jnp.int32, x2.shape, 1)
    x2 = jnp.where(col < 268, x2, 0.0)
    g = lax.dot_general(x2, x2, (((0,), (0,)), ((), ())),
                        preferred_element_type=jnp.float32)[:268, :268]

    @pl.when(i == 0)
    def _():
        out_ref[...] = g

    @pl.when(i > 0)
    def _():
        out_ref[...] += g


def _tc_final(t0, t1, degp, b2, g2, be2):
    return pl.pallas_call(
        _final_body,
        grid=(N // _BS,),
        in_specs=[
            pl.BlockSpec((_BS, W2H), lambda i: (i, 0)),
            pl.BlockSpec((_BS, W2H), lambda i: (i, 0)),
            pl.BlockSpec((NC, _BS, 16), lambda i: (0, i, 0)),
            pl.BlockSpec((268,), lambda i: (0,)),
            pl.BlockSpec((268,), lambda i: (0,)),
            pl.BlockSpec((268,), lambda i: (0,)),
        ],
        out_specs=pl.BlockSpec((268, 268), lambda i: (0, 0)),
        out_shape=jax.ShapeDtypeStruct((268, 268), jnp.float32),
    )(t0, t1, degp, b2, g2, be2)


def kernel(x, edge_index, edge_attr, W1, b1, gamma1, beta1, W2, b2, gamma2,
           beta2):
    xs = x.reshape(N, 160)
    e4 = edge_index.reshape(2, E // CH, CH)
    e50 = edge_index.reshape(2, E // 50, 50)
    zeros_n = jnp.zeros((NPAD, 16), jnp.float32)
    ones_n = jnp.ones((128, 16), jnp.float32)

    degp = _deg_kernel(e4, zeros_n, ones_n)
    v0, v1 = _tc_scale(xs, degp)
    s0, s1 = _prop80(e4, v0, v1)
    h0, h1 = _tc_mid(s0, s1, degp, W1, b1, gamma1, beta1, W2)
    t0, t1 = _prop144(e50, h0, h1)
    return _tc_final(t0, t1, degp, b2, gamma2, beta2)
